# baseline (speedup 1.0000x reference)
import functools

import jax
import jax.numpy as jnp
from jax import lax
from jax.experimental import pallas as pl
from jax.experimental.pallas import tpu as pltpu
from jax.experimental.pallas import tpu_sc as plsc

_NBATCH, _NPATCH, _DPATCH, _NMIC = 32, 512, 256, 4
_NMASKED = 100
_SUB, _LANE = 8, 128

_SC_LANES = 16
_IDX_PAD = 112
_CHUNKS = _IDX_PAD // _SC_LANES
_WORKERS = 32
_TILE_P = _NPATCH // _WORKERS
_TILE_ELEMS = _TILE_P * _LANE
_SENTINEL = 1 << 20


def _flags_sc_body(idx_hbm, flags_hbm, idx_v, row_v):
    info = plsc.get_sparse_core_info()
    wid = lax.axis_index("s") * info.num_cores + lax.axis_index("c")

    pltpu.sync_copy(idx_hbm.at[pl.ds(wid * _IDX_PAD, _IDX_PAD)], idx_v)

    ones = jnp.ones((_SC_LANES,), jnp.float32)
    for off in range(0, _NPATCH, _SC_LANES):
        row_v[pl.ds(off, _SC_LANES)] = ones

    zeros = jnp.zeros((_SC_LANES,), jnp.float32)
    for k in range(_CHUNKS):
        chunk = idx_v[pl.ds(k * _SC_LANES, _SC_LANES)]
        plsc.store_scatter(row_v, [chunk], zeros, mask=chunk < _NPATCH)

    pltpu.sync_copy(row_v, flags_hbm.at[pl.ds(wid * _NPATCH, _NPATCH)])


_flags_sc = functools.partial(
    pl.kernel,
    out_type=jax.ShapeDtypeStruct((_NBATCH * _NPATCH,), jnp.float32),
    mesh=plsc.VectorSubcoreMesh(core_axis_name="c", subcore_axis_name="s"),
    scratch_types=[
        pltpu.VMEM((_IDX_PAD,), jnp.int32),
        pltpu.VMEM((_NPATCH,), jnp.float32),
    ],
    compiler_params=pltpu.CompilerParams(needs_layout_passes=False),
)(_flags_sc_body)


def _ch_block(ch_ref, b):
    c = ch_ref[b, 0]
    shape = (_NPATCH, _SUB, _LANE)
    sub_ch = jax.lax.broadcasted_iota(jnp.int32, shape, 1) % _NMIC
    return jnp.where(sub_ch == c, 0.0, 1.0).astype(jnp.float32)


def _chm_kernel(ch_ref, chm_ref):
    chm_ref[...] = _ch_block(ch_ref, pl.program_id(0))


def _masked_kernel(flags_ref, ch_ref, dense_ref, patch_ref):
    b = pl.program_id(0)
    shape = (_NPATCH, _SUB, _LANE)
    col = jnp.transpose(flags_ref[0], (1, 0))
    patch_block = jnp.broadcast_to(col[:, :, None], shape)

    patch_ref[...] = patch_block
    dense_ref[...] = jnp.maximum(patch_block, _ch_block(ch_ref, b))


def kernel(base, mask_patch_idx, mask_ch_idx):
    del base
    idx_flat = jnp.pad(
        mask_patch_idx, ((0, 0), (0, _IDX_PAD - _NMASKED)),
        constant_values=_SENTINEL,
    ).reshape(_NBATCH * _IDX_PAD)

    flags = _flags_sc(idx_flat).reshape(_NBATCH, 1, _NPATCH)

    rows = _NBATCH * _NPATCH
    out_shape = jax.ShapeDtypeStruct((rows, _SUB, _LANE), jnp.float32)
    block = pl.BlockSpec((_NPATCH, _SUB, _LANE), lambda b: (b, 0, 0))
    chm = pl.pallas_call(
        _chm_kernel,
        grid=(_NBATCH,),
        in_specs=[pl.BlockSpec(memory_space=pltpu.SMEM)],
        out_specs=block,
        out_shape=out_shape,
    )(mask_ch_idx)
    dense, patch = pl.pallas_call(
        _masked_kernel,
        grid=(_NBATCH,),
        in_specs=[
            pl.BlockSpec((1, 1, _NPATCH), lambda b: (b, 0, 0)),
            pl.BlockSpec(memory_space=pltpu.SMEM),
        ],
        out_specs=[block, block],
        out_shape=[out_shape, out_shape],
    )(flags, mask_ch_idx)

    def to_logical(a):
        a = a.reshape(_NBATCH, _NPATCH, 2, _NMIC, _LANE)
        a = a.transpose(0, 1, 2, 4, 3)
        return a.reshape(_NBATCH, _NPATCH, _DPATCH, _NMIC)

    return (
        to_logical(dense),
        to_logical(patch),
        to_logical(chm),
        mask_patch_idx,
        mask_ch_idx,
    )


# speedup vs baseline: 1.0203x; 1.0203x over previous
import functools

import jax
import jax.numpy as jnp
from jax import lax
from jax.experimental import pallas as pl
from jax.experimental.pallas import tpu as pltpu
from jax.experimental.pallas import tpu_sc as plsc

_NBATCH, _NPATCH, _DPATCH, _NMIC = 32, 512, 256, 4
_NMASKED = 100
_SUB, _LANE = 8, 128

_SC_LANES = 16
_IDX_PAD = 112
_CHUNKS = _IDX_PAD // _SC_LANES
_WORKERS = 32
_TILE_P = _NPATCH // _WORKERS
_TILE_ELEMS = _TILE_P * _LANE
_SENTINEL = 1 << 20


def _flags_sc_body(idx_hbm, flags_hbm, idx_v, row_v):
    info = plsc.get_sparse_core_info()
    wid = lax.axis_index("s") * info.num_cores + lax.axis_index("c")

    pltpu.sync_copy(idx_hbm.at[pl.ds(wid * _IDX_PAD, _IDX_PAD)], idx_v)

    ones = jnp.ones((_SC_LANES,), jnp.float32)
    for off in range(0, _NPATCH, _SC_LANES):
        row_v[pl.ds(off, _SC_LANES)] = ones

    zeros = jnp.zeros((_SC_LANES,), jnp.float32)
    for k in range(_CHUNKS):
        chunk = idx_v[pl.ds(k * _SC_LANES, _SC_LANES)]
        plsc.store_scatter(row_v, [chunk], zeros, mask=chunk < _NPATCH)

    pltpu.sync_copy(row_v, flags_hbm.at[pl.ds(wid * _NPATCH, _NPATCH)])


_flags_sc = functools.partial(
    pl.kernel,
    out_type=jax.ShapeDtypeStruct((_NBATCH * _NPATCH,), jnp.float32),
    mesh=plsc.VectorSubcoreMesh(core_axis_name="c", subcore_axis_name="s"),
    scratch_types=[
        pltpu.VMEM((_IDX_PAD,), jnp.int32),
        pltpu.VMEM((_NPATCH,), jnp.float32),
    ],
    compiler_params=pltpu.CompilerParams(needs_layout_passes=False),
)(_flags_sc_body)


def _ch_block(ch_ref, b):
    c = ch_ref[b, 0]
    shape = (_NPATCH, _SUB, _LANE)
    sub_ch = jax.lax.broadcasted_iota(jnp.int32, shape, 1) % _NMIC
    return jnp.where(sub_ch == c, 0.0, 1.0).astype(jnp.float32)


def _mask_kernel(flags_ref, ch_ref, dense_ref, patch_ref, chm_ref):
    b = pl.program_id(0)
    shape = (_NPATCH, _SUB, _LANE)
    col = jnp.transpose(flags_ref[0], (1, 0))
    patch_block = jnp.broadcast_to(col[:, :, None], shape)
    ch_block = _ch_block(ch_ref, b)

    patch_ref[...] = patch_block
    chm_ref[...] = ch_block
    dense_ref[...] = jnp.maximum(patch_block, ch_block)


def kernel(base, mask_patch_idx, mask_ch_idx):
    del base
    idx_flat = jnp.pad(
        mask_patch_idx, ((0, 0), (0, _IDX_PAD - _NMASKED)),
        constant_values=_SENTINEL,
    ).reshape(_NBATCH * _IDX_PAD)

    flags = _flags_sc(idx_flat).reshape(_NBATCH, 1, _NPATCH)

    rows = _NBATCH * _NPATCH
    out_shape = jax.ShapeDtypeStruct((rows, _SUB, _LANE), jnp.float32)
    block = pl.BlockSpec((_NPATCH, _SUB, _LANE), lambda b: (b, 0, 0))
    dense, patch, chm = pl.pallas_call(
        _mask_kernel,
        grid=(_NBATCH,),
        in_specs=[
            pl.BlockSpec((1, 1, _NPATCH), lambda b: (b, 0, 0)),
            pl.BlockSpec(memory_space=pltpu.SMEM),
        ],
        out_specs=[block, block, block],
        out_shape=[out_shape, out_shape, out_shape],
    )(flags, mask_ch_idx)

    def to_logical(a):
        a = a.reshape(_NBATCH, _NPATCH, 2, _NMIC, _LANE)
        a = a.transpose(0, 1, 2, 4, 3)
        return a.reshape(_NBATCH, _NPATCH, _DPATCH, _NMIC)

    return (
        to_logical(dense),
        to_logical(patch),
        to_logical(chm),
        mask_patch_idx,
        mask_ch_idx,
    )


# speedup vs baseline: 1.0207x; 1.0004x over previous
import functools

import jax
import jax.numpy as jnp
from jax import lax
from jax.experimental import pallas as pl
from jax.experimental.pallas import tpu as pltpu
from jax.experimental.pallas import tpu_sc as plsc

_NBATCH, _NPATCH, _DPATCH, _NMIC = 32, 512, 256, 4
_NMASKED = 100
_SUB, _LANE = 8, 128

_SC_LANES = 16
_IDX_PAD = 112
_CHUNKS = _IDX_PAD // _SC_LANES
_WORKERS = 32
_TILE_P = _NPATCH // _WORKERS
_TILE_ELEMS = _TILE_P * _LANE
_SENTINEL = 1 << 20


def _flags_sc_body(idx_hbm, flags_hbm, idx_v, row_v):
    info = plsc.get_sparse_core_info()
    wid = lax.axis_index("s") * info.num_cores + lax.axis_index("c")

    pltpu.sync_copy(idx_hbm.at[pl.ds(wid * _IDX_PAD, _IDX_PAD)], idx_v)

    ones = jnp.ones((_SC_LANES,), jnp.float32)
    for off in range(0, _NPATCH, _SC_LANES):
        row_v[pl.ds(off, _SC_LANES)] = ones

    zeros = jnp.zeros((_SC_LANES,), jnp.float32)
    for k in range(_CHUNKS):
        chunk = idx_v[pl.ds(k * _SC_LANES, _SC_LANES)]
        plsc.store_scatter(row_v, [chunk], zeros, mask=chunk < _NPATCH)

    pltpu.sync_copy(row_v, flags_hbm.at[pl.ds(wid * _NPATCH, _NPATCH)])


_flags_sc = functools.partial(
    pl.kernel,
    out_type=jax.ShapeDtypeStruct((_NBATCH * _NPATCH,), jnp.float32),
    mesh=plsc.VectorSubcoreMesh(core_axis_name="c", subcore_axis_name="s"),
    scratch_types=[
        pltpu.VMEM((_IDX_PAD,), jnp.int32),
        pltpu.VMEM((_NPATCH,), jnp.float32),
    ],
    compiler_params=pltpu.CompilerParams(
        needs_layout_passes=False,
        skip_device_barrier=True,
        disable_bounds_checks=True,
        disable_semaphore_checks=True,
    ),
)(_flags_sc_body)


def _ch_block(ch_ref, b):
    c = ch_ref[b, 0]
    shape = (_NPATCH, _SUB, _LANE)
    sub_ch = jax.lax.broadcasted_iota(jnp.int32, shape, 1) % _NMIC
    return jnp.where(sub_ch == c, 0.0, 1.0).astype(jnp.float32)


def _mask_kernel(flags_ref, ch_ref, dense_ref, patch_ref, chm_ref):
    b = pl.program_id(0)
    shape = (_NPATCH, _SUB, _LANE)
    col = jnp.transpose(flags_ref[0], (1, 0))
    patch_block = jnp.broadcast_to(col[:, :, None], shape)
    ch_block = _ch_block(ch_ref, b)

    patch_ref[...] = patch_block
    chm_ref[...] = ch_block
    dense_ref[...] = jnp.maximum(patch_block, ch_block)


def kernel(base, mask_patch_idx, mask_ch_idx):
    del base
    idx_flat = jnp.pad(
        mask_patch_idx, ((0, 0), (0, _IDX_PAD - _NMASKED)),
        constant_values=_SENTINEL,
    ).reshape(_NBATCH * _IDX_PAD)

    flags = _flags_sc(idx_flat).reshape(_NBATCH, 1, _NPATCH)

    rows = _NBATCH * _NPATCH
    out_shape = jax.ShapeDtypeStruct((rows, _SUB, _LANE), jnp.float32)
    block = pl.BlockSpec((_NPATCH, _SUB, _LANE), lambda b: (b, 0, 0))
    dense, patch, chm = pl.pallas_call(
        _mask_kernel,
        grid=(_NBATCH,),
        in_specs=[
            pl.BlockSpec((1, 1, _NPATCH), lambda b: (b, 0, 0)),
            pl.BlockSpec(memory_space=pltpu.SMEM),
        ],
        out_specs=[block, block, block],
        out_shape=[out_shape, out_shape, out_shape],
    )(flags, mask_ch_idx)

    def to_logical(a):
        a = a.reshape(_NBATCH, _NPATCH, 2, _NMIC, _LANE)
        a = a.transpose(0, 1, 2, 4, 3)
        return a.reshape(_NBATCH, _NPATCH, _DPATCH, _NMIC)

    return (
        to_logical(dense),
        to_logical(patch),
        to_logical(chm),
        mask_patch_idx,
        mask_ch_idx,
    )
